# initial kernel scaffold (unmeasured)
import jax
import jax.numpy as jnp
from jax import lax
from jax.experimental import pallas as pl
from jax.experimental.pallas import tpu as pltpu

N_DEV = 4


def kernel(x, w_mat, scale_x, scale_w):
    m_per, k = x.shape
    n = w_mat.shape[1]
    n_per = n // N_DEV
    my = lax.axis_index("i")

    x8 = x.astype(jnp.float8_e4m3fn)
    w_loc = lax.dynamic_slice_in_dim(w_mat, my * n_per, n_per, axis=1)
    w8 = w_loc.astype(jnp.float8_e5m2)

    def body(x_ref, w_ref, sx_ref, sw_ref, out_ref, comm_ref, send_sems, recv_sems):
        my_pos = lax.axis_index("i")
        left = (my_pos - 1) % N_DEV
        right = (my_pos + 1) % N_DEV

        barrier = pltpu.get_barrier_semaphore()
        for nbr in (left, right):
            pl.semaphore_signal(
                barrier, inc=1,
                device_id=(nbr,), device_id_type=pl.DeviceIdType.MESH,
            )
        pl.semaphore_wait(barrier, 2)

        s = sx_ref[0] * sw_ref[0]

        def block(a8, origin):
            acc = lax.dot_general(
                a8, w_ref[...], (((1,), (0,)), ((), ())),
                preferred_element_type=jnp.float32,
            )
            y = acc * s
            out_ref[pl.ds(origin * m_per, m_per), :] = y * jax.nn.sigmoid(y)

        for h in range(N_DEV - 1):
            src = x_ref if h == 0 else comm_ref.at[h - 1]
            rdma = pltpu.make_async_remote_copy(
                src_ref=src,
                dst_ref=comm_ref.at[h],
                send_sem=send_sems.at[h],
                recv_sem=recv_sems.at[h],
                device_id=(right,),
                device_id_type=pl.DeviceIdType.MESH,
            )
            rdma.start()
            if h == 0:
                block(x_ref[...], my_pos)
            else:
                block(comm_ref[h - 1], (my_pos - h) % N_DEV)
            rdma.wait()
        block(comm_ref[N_DEV - 2], (my_pos - (N_DEV - 1)) % N_DEV)

    return pl.pallas_call(
        body,
        out_shape=jax.ShapeDtypeStruct((N_DEV * m_per, n_per), jnp.float32),
        in_specs=[
            pl.BlockSpec(memory_space=pltpu.VMEM),
            pl.BlockSpec(memory_space=pltpu.VMEM),
            pl.BlockSpec(memory_space=pltpu.SMEM),
            pl.BlockSpec(memory_space=pltpu.SMEM),
        ],
        out_specs=pl.BlockSpec(memory_space=pltpu.VMEM),
        scratch_shapes=[
            pltpu.VMEM((N_DEV - 1, m_per, k), jnp.float8_e4m3fn),
            pltpu.SemaphoreType.DMA((N_DEV - 1,)),
            pltpu.SemaphoreType.DMA((N_DEV - 1,)),
        ],
        compiler_params=pltpu.CompilerParams(collective_id=0),
    )(x8, w8, scale_x, scale_w)


# baseline (device time: 218866 ns/iter reference)
import jax
import jax.numpy as jnp
from jax import lax
from jax.experimental import pallas as pl
from jax.experimental.pallas import tpu as pltpu

N_DEV = 4


def kernel(x, w_mat, scale_x, scale_w):
    m_per, k = x.shape
    n = w_mat.shape[1]
    n_per = n // N_DEV
    my = lax.axis_index("i")

    x8 = x.astype(jnp.float8_e4m3fn)
    w_loc = lax.dynamic_slice_in_dim(w_mat, my * n_per, n_per, axis=1)
    w8 = w_loc.astype(jnp.float8_e5m2)

    def body(x_ref, w_ref, sx_ref, sw_ref, out_ref, comm_ref, send_sems, recv_sems):
        my_pos = lax.axis_index("i")
        left = (my_pos - 1) % N_DEV
        right = (my_pos + 1) % N_DEV

        barrier = pltpu.get_barrier_semaphore()
        for nbr in (left, right):
            pl.semaphore_signal(
                barrier, inc=1,
                device_id=(nbr,), device_id_type=pl.DeviceIdType.MESH,
            )
        pl.semaphore_wait(barrier, 2)

        s = sx_ref[0] * sw_ref[0]

        def block(a8, origin):
            acc = lax.dot_general(
                a8, w_ref[...], (((1,), (0,)), ((), ())),
                preferred_element_type=jnp.float32,
            )
            y = acc * s
            out_ref[pl.ds(origin * m_per, m_per), :] = y * jax.nn.sigmoid(y)

        for h in range(N_DEV - 1):
            src = x_ref if h == 0 else comm_ref.at[h - 1]
            rdma = pltpu.make_async_remote_copy(
                src_ref=src,
                dst_ref=comm_ref.at[h],
                send_sem=send_sems.at[h],
                recv_sem=recv_sems.at[h],
                device_id=(right,),
                device_id_type=pl.DeviceIdType.MESH,
            )
            rdma.start()
            if h == 0:
                block(x_ref[...], my_pos)
            else:
                block(comm_ref[h - 1], (my_pos - h) % N_DEV)
            rdma.wait()
        block(comm_ref[N_DEV - 2], (my_pos - (N_DEV - 1)) % N_DEV)

    return pl.pallas_call(
        body,
        out_shape=jax.ShapeDtypeStruct((N_DEV * m_per, n_per), jnp.float32),
        in_specs=[
            pl.BlockSpec(memory_space=pltpu.VMEM),
            pl.BlockSpec(memory_space=pltpu.VMEM),
            pl.BlockSpec(memory_space=pltpu.SMEM),
            pl.BlockSpec(memory_space=pltpu.SMEM),
        ],
        out_specs=pl.BlockSpec(memory_space=pltpu.VMEM),
        scratch_shapes=[
            pltpu.VMEM((N_DEV - 1, m_per, k), jnp.float8_e4m3fn),
            pltpu.SemaphoreType.DMA((N_DEV - 1,)),
            pltpu.SemaphoreType.DMA((N_DEV - 1,)),
        ],
        compiler_params=pltpu.CompilerParams(
            collective_id=0, vmem_limit_bytes=110 * 1024 * 1024
        ),
    )(x8, w8, scale_x, scale_w)


# device time: 151585 ns/iter; 1.4438x vs baseline; 1.4438x over previous
import jax
import jax.numpy as jnp
from jax import lax
from jax.experimental import pallas as pl
from jax.experimental.pallas import tpu as pltpu

N_DEV = 4


def kernel(x, w_mat, scale_x, scale_w):
    m_per, k = x.shape
    n = w_mat.shape[1]
    n_per = n // N_DEV
    half = m_per // 2
    my = lax.axis_index("i")

    x8 = x.astype(jnp.float8_e4m3fn)
    w_loc = lax.dynamic_slice_in_dim(w_mat, my * n_per, n_per, axis=1)
    w8 = w_loc.astype(jnp.float8_e5m2)

    def body(x_ref, w_ref, sx_ref, sw_ref, out_ref,
             comm_r, comm_l, send_r, recv_r, send_l, recv_l):
        my_pos = lax.axis_index("i")
        left = (my_pos - 1) % N_DEV
        right = (my_pos + 1) % N_DEV

        barrier = pltpu.get_barrier_semaphore()
        for nbr in (left, right):
            pl.semaphore_signal(
                barrier, inc=1,
                device_id=(nbr,), device_id_type=pl.DeviceIdType.MESH,
            )
        pl.semaphore_wait(barrier, 2)

        s = sx_ref[0] * sw_ref[0]

        def block(a8, origin, row_off, rows):
            acc = lax.dot_general(
                a8, w_ref[...], (((1,), (0,)), ((), ())),
                preferred_element_type=jnp.float32,
            )
            y = acc * s
            out_ref[pl.ds(origin * m_per + row_off, rows), :] = (
                y * jax.nn.sigmoid(y)
            )

        for h in range(N_DEV - 1):
            src_r = x_ref.at[pl.ds(0, half)] if h == 0 else comm_r.at[h - 1]
            src_l = x_ref.at[pl.ds(half, half)] if h == 0 else comm_l.at[h - 1]
            rdma_r = pltpu.make_async_remote_copy(
                src_ref=src_r, dst_ref=comm_r.at[h],
                send_sem=send_r.at[h], recv_sem=recv_r.at[h],
                device_id=(right,), device_id_type=pl.DeviceIdType.MESH,
            )
            rdma_l = pltpu.make_async_remote_copy(
                src_ref=src_l, dst_ref=comm_l.at[h],
                send_sem=send_l.at[h], recv_sem=recv_l.at[h],
                device_id=(left,), device_id_type=pl.DeviceIdType.MESH,
            )
            rdma_r.start()
            rdma_l.start()
            if h == 0:
                block(x_ref[...], my_pos, 0, m_per)
            else:
                block(comm_r[h - 1], (my_pos - h) % N_DEV, 0, half)
                block(comm_l[h - 1], (my_pos + h) % N_DEV, half, half)
            rdma_r.wait()
            rdma_l.wait()
        block(comm_r[N_DEV - 2], (my_pos + 1) % N_DEV, 0, half)
        block(comm_l[N_DEV - 2], (my_pos - 1) % N_DEV, half, half)

    return pl.pallas_call(
        body,
        out_shape=jax.ShapeDtypeStruct((N_DEV * m_per, n_per), jnp.float32),
        in_specs=[
            pl.BlockSpec(memory_space=pltpu.VMEM),
            pl.BlockSpec(memory_space=pltpu.VMEM),
            pl.BlockSpec(memory_space=pltpu.SMEM),
            pl.BlockSpec(memory_space=pltpu.SMEM),
        ],
        out_specs=pl.BlockSpec(memory_space=pltpu.VMEM),
        scratch_shapes=[
            pltpu.VMEM((N_DEV - 1, half, k), jnp.float8_e4m3fn),
            pltpu.VMEM((N_DEV - 1, half, k), jnp.float8_e4m3fn),
            pltpu.SemaphoreType.DMA((N_DEV - 1,)),
            pltpu.SemaphoreType.DMA((N_DEV - 1,)),
            pltpu.SemaphoreType.DMA((N_DEV - 1,)),
            pltpu.SemaphoreType.DMA((N_DEV - 1,)),
        ],
        compiler_params=pltpu.CompilerParams(
            collective_id=0, vmem_limit_bytes=110 * 1024 * 1024
        ),
    )(x8, w8, scale_x, scale_w)


# device time: 120735 ns/iter; 1.8128x vs baseline; 1.2555x over previous
import jax
import jax.numpy as jnp
from jax import lax
from jax.experimental import pallas as pl
from jax.experimental.pallas import tpu as pltpu

N_DEV = 4


def kernel(x, w_mat, scale_x, scale_w):
    m_per, k = x.shape
    n = w_mat.shape[1]
    n_per = n // N_DEV
    half = m_per // 2
    wq = 4
    nq = n_per // wq

    def body(x_hbm, w_hbm, sx_ref, sw_ref, out_hbm,
             xf_ref, x8_ref, wf_ref, w8_ref, comm_r, comm_l, ostage,
             send_r, recv_r, send_l, recv_l, xf_sem, wf_sem, os_sem):
        my_pos = lax.axis_index("i")
        left = (my_pos - 1) % N_DEV
        right = (my_pos + 1) % N_DEV

        barrier = pltpu.get_barrier_semaphore()
        for nbr in (left, right):
            pl.semaphore_signal(
                barrier, inc=1,
                device_id=(nbr,), device_id_type=pl.DeviceIdType.MESH,
            )
        pl.semaphore_wait(barrier, 2)

        def hop_start(h, direction):
            if direction == "r":
                src = x8_ref.at[pl.ds(0, half)] if h == 0 else comm_r.at[h - 1]
                rdma = pltpu.make_async_remote_copy(
                    src_ref=src, dst_ref=comm_r.at[h],
                    send_sem=send_r.at[h], recv_sem=recv_r.at[h],
                    device_id=(right,), device_id_type=pl.DeviceIdType.MESH,
                )
            else:
                src = x8_ref.at[pl.ds(half, half)] if h == 0 else comm_l.at[h - 1]
                rdma = pltpu.make_async_remote_copy(
                    src_ref=src, dst_ref=comm_l.at[h],
                    send_sem=send_l.at[h], recv_sem=recv_l.at[h],
                    device_id=(left,), device_id_type=pl.DeviceIdType.MESH,
                )
            rdma.start()
            return rdma

        cp = pltpu.make_async_copy(x_hbm.at[pl.ds(0, half)], xf_ref, xf_sem)
        cp.start()
        cp.wait()
        x8_ref[pl.ds(0, half), :] = xf_ref[...].astype(jnp.float8_e4m3fn)
        rdma_r = hop_start(0, "r")
        cp = pltpu.make_async_copy(x_hbm.at[pl.ds(half, half)], xf_ref, xf_sem)
        cp.start()
        cp.wait()
        x8_ref[pl.ds(half, half), :] = xf_ref[...].astype(jnp.float8_e4m3fn)
        rdma_l = hop_start(0, "l")

        for q in range(wq):
            cp = pltpu.make_async_copy(
                w_hbm.at[:, pl.ds(my_pos * n_per + q * nq, nq)], wf_ref, wf_sem,
            )
            cp.start()
            cp.wait()
            w8_ref[:, pl.ds(q * nq, nq)] = wf_ref[...].astype(jnp.float8_e5m2)

        s = sx_ref[0] * sw_ref[0]

        pending = [None, None]
        counter = [0]

        def block_half(a8, origin, off):
            acc = lax.dot_general(
                a8, w8_ref[...], (((1,), (0,)), ((), ())),
                preferred_element_type=jnp.float32,
            )
            y = acc * s
            slot = counter[0] % 2
            if pending[slot] is not None:
                pending[slot].wait()
            ostage[slot, :, :] = y * jax.nn.sigmoid(y)
            c = pltpu.make_async_copy(
                ostage.at[slot],
                out_hbm.at[pl.ds(origin * m_per + off, half)],
                os_sem.at[slot],
            )
            c.start()
            pending[slot] = c
            counter[0] += 1

        block_half(x8_ref[pl.ds(0, half), :], my_pos, 0)
        rdma_r.wait()
        rdma_l.wait()
        rdma_r = hop_start(1, "r")
        rdma_l = hop_start(1, "l")
        block_half(x8_ref[pl.ds(half, half), :], my_pos, half)
        block_half(comm_r[0], (my_pos - 1) % N_DEV, 0)
        block_half(comm_l[0], (my_pos + 1) % N_DEV, half)
        rdma_r.wait()
        rdma_l.wait()
        rdma_r = hop_start(2, "r")
        rdma_l = hop_start(2, "l")
        block_half(comm_r[1], (my_pos - 2) % N_DEV, 0)
        block_half(comm_l[1], (my_pos + 2) % N_DEV, half)
        rdma_r.wait()
        rdma_l.wait()
        block_half(comm_r[2], (my_pos + 1) % N_DEV, 0)
        block_half(comm_l[2], (my_pos - 1) % N_DEV, half)
        for p in pending:
            p.wait()

    return pl.pallas_call(
        body,
        out_shape=jax.ShapeDtypeStruct((N_DEV * m_per, n_per), jnp.float32),
        in_specs=[
            pl.BlockSpec(memory_space=pltpu.HBM),
            pl.BlockSpec(memory_space=pltpu.HBM),
            pl.BlockSpec(memory_space=pltpu.SMEM),
            pl.BlockSpec(memory_space=pltpu.SMEM),
        ],
        out_specs=pl.BlockSpec(memory_space=pltpu.HBM),
        scratch_shapes=[
            pltpu.VMEM((half, k), jnp.float32),
            pltpu.VMEM((m_per, k), jnp.float8_e4m3fn),
            pltpu.VMEM((k, nq), jnp.float32),
            pltpu.VMEM((k, n_per), jnp.float8_e5m2),
            pltpu.VMEM((N_DEV - 1, half, k), jnp.float8_e4m3fn),
            pltpu.VMEM((N_DEV - 1, half, k), jnp.float8_e4m3fn),
            pltpu.VMEM((2, half, n_per), jnp.float32),
            pltpu.SemaphoreType.DMA((N_DEV - 1,)),
            pltpu.SemaphoreType.DMA((N_DEV - 1,)),
            pltpu.SemaphoreType.DMA((N_DEV - 1,)),
            pltpu.SemaphoreType.DMA((N_DEV - 1,)),
            pltpu.SemaphoreType.DMA,
            pltpu.SemaphoreType.DMA,
            pltpu.SemaphoreType.DMA((2,)),
        ],
        compiler_params=pltpu.CompilerParams(
            collective_id=0, vmem_limit_bytes=63 * 1024 * 1024
        ),
    )(x, w_mat, scale_x, scale_w)


# device time: 110618 ns/iter; 1.9786x vs baseline; 1.0915x over previous
import contextlib
import os

import jax
import jax.numpy as jnp
from jax import lax
from jax.experimental import pallas as pl
from jax.experimental.pallas import tpu as pltpu

N_DEV = 4
N_SUB = 2
_PROF = os.environ.get("KPROF") == "1"


def _scope(name):
    return jax.named_scope(name) if _PROF else contextlib.nullcontext()


def kernel(x, w_mat, scale_x, scale_w):
    m_per, k = x.shape
    n = w_mat.shape[1]
    n_per = n // N_DEV
    half = m_per // 2
    m_sub = half // N_SUB
    wq = 4
    nq = n_per // wq

    def body(x_hbm, w_hbm, sx_ref, sw_ref, out_hbm,
             xf_ref, x8_ref, wf_ref, w8_ref, comm_r, comm_l, ostage,
             send_r, recv_r, send_l, recv_l, xf_sem, wf_sem, os_sem):
        my_pos = lax.axis_index("i")
        left = (my_pos - 1) % N_DEV
        right = (my_pos + 1) % N_DEV

        barrier = pltpu.get_barrier_semaphore()
        for nbr in (left, right):
            pl.semaphore_signal(
                barrier, inc=1,
                device_id=(nbr,), device_id_type=pl.DeviceIdType.MESH,
            )
        pl.semaphore_wait(barrier, 2)

        def rdma(h, s, direction):
            rows = pl.ds(s * m_sub, m_sub)
            if direction == "r":
                src = (x8_ref.at[pl.ds(s * m_sub, m_sub)] if h == 0
                       else comm_r.at[h - 1, rows])
                return pltpu.make_async_remote_copy(
                    src_ref=src, dst_ref=comm_r.at[h, rows],
                    send_sem=send_r.at[h, s], recv_sem=recv_r.at[h, s],
                    device_id=(right,), device_id_type=pl.DeviceIdType.MESH,
                )
            src = (x8_ref.at[pl.ds(half + s * m_sub, m_sub)] if h == 0
                   else comm_l.at[h - 1, rows])
            return pltpu.make_async_remote_copy(
                src_ref=src, dst_ref=comm_l.at[h, rows],
                send_sem=send_l.at[h, s], recv_sem=recv_l.at[h, s],
                device_id=(left,), device_id_type=pl.DeviceIdType.MESH,
            )

        live = {}

        def start(h, s, d):
            r = rdma(h, s, d)
            r.start()
            live[(h, s, d)] = r

        with _scope("x_prep"):
            cp = pltpu.make_async_copy(x_hbm.at[pl.ds(0, half)], xf_ref, xf_sem)
            cp.start()
            cp.wait()
            x8_ref[pl.ds(0, half), :] = xf_ref[...].astype(jnp.float8_e4m3fn)
            start(0, 0, "r")
            start(0, 1, "r")
            cp = pltpu.make_async_copy(
                x_hbm.at[pl.ds(half, half)], xf_ref, xf_sem,
            )
            cp.start()
            cp.wait()
            x8_ref[pl.ds(half, half), :] = xf_ref[...].astype(jnp.float8_e4m3fn)
            start(0, 0, "l")
            start(0, 1, "l")

        with _scope("w_prep"):
            wcp = [None, None]
            for q in range(wq):
                c = pltpu.make_async_copy(
                    w_hbm.at[:, pl.ds(my_pos * n_per + q * nq, nq)],
                    wf_ref.at[q % 2], wf_sem.at[q % 2],
                )
                c.start()
                wcp[q % 2] = c
                if q >= 1:
                    wcp[(q - 1) % 2].wait()
                    w8_ref[:, pl.ds((q - 1) * nq, nq)] = (
                        wf_ref[(q - 1) % 2].astype(jnp.float8_e5m2)
                    )
            wcp[(wq - 1) % 2].wait()
            w8_ref[:, pl.ds((wq - 1) * nq, nq)] = (
                wf_ref[(wq - 1) % 2].astype(jnp.float8_e5m2)
            )

        s_scale = sx_ref[0] * sw_ref[0]

        pending = [None, None]
        counter = [0]

        def block(a8, origin, off):
            acc = lax.dot_general(
                a8, w8_ref[...], (((1,), (0,)), ((), ())),
                preferred_element_type=jnp.float32,
            )
            y = acc * s_scale
            slot = counter[0] % 2
            if pending[slot] is not None:
                pending[slot].wait()
            ostage[slot, :, :] = y * jax.nn.sigmoid(y)
            c = pltpu.make_async_copy(
                ostage.at[slot],
                out_hbm.at[pl.ds(origin * m_per + off, m_sub)],
                os_sem.at[slot],
            )
            c.start()
            pending[slot] = c
            counter[0] += 1

        def sub_r(h, s):
            block(comm_r[h, pl.ds(s * m_sub, m_sub)],
                  (my_pos - h - 1) % N_DEV, s * m_sub)

        def sub_l(h, s):
            block(comm_l[h, pl.ds(s * m_sub, m_sub)],
                  (my_pos + h + 1) % N_DEV, half + s * m_sub)

        with _scope("gemm_own"):
            for r in range(4):
                block(x8_ref[pl.ds(r * m_sub, m_sub), :], my_pos, r * m_sub)

        for h in range(N_DEV - 2):
            for s in range(N_SUB):
                with _scope(f"wait_h{h}s{s}"):
                    live.pop((h, s, "r")).wait()
                    live.pop((h, s, "l")).wait()
                start(h + 1, s, "r")
                start(h + 1, s, "l")
                with _scope(f"gemm_h{h}s{s}"):
                    sub_r(h, s)
                    sub_l(h, s)
        h_last = N_DEV - 2
        for s in range(N_SUB):
            with _scope(f"wait_h{h_last}s{s}"):
                live.pop((h_last, s, "r")).wait()
                live.pop((h_last, s, "l")).wait()
            with _scope(f"gemm_h{h_last}s{s}"):
                sub_r(h_last, s)
                sub_l(h_last, s)
        with _scope("flush"):
            for p in pending:
                p.wait()

    return pl.pallas_call(
        body,
        out_shape=jax.ShapeDtypeStruct((N_DEV * m_per, n_per), jnp.float32),
        in_specs=[
            pl.BlockSpec(memory_space=pltpu.HBM),
            pl.BlockSpec(memory_space=pltpu.HBM),
            pl.BlockSpec(memory_space=pltpu.SMEM),
            pl.BlockSpec(memory_space=pltpu.SMEM),
        ],
        out_specs=pl.BlockSpec(memory_space=pltpu.HBM),
        scratch_shapes=[
            pltpu.VMEM((half, k), jnp.float32),
            pltpu.VMEM((m_per, k), jnp.float8_e4m3fn),
            pltpu.VMEM((2, k, nq), jnp.float32),
            pltpu.VMEM((k, n_per), jnp.float8_e5m2),
            pltpu.VMEM((N_DEV - 1, half, k), jnp.float8_e4m3fn),
            pltpu.VMEM((N_DEV - 1, half, k), jnp.float8_e4m3fn),
            pltpu.VMEM((2, m_sub, n_per), jnp.float32),
            pltpu.SemaphoreType.DMA((N_DEV - 1, N_SUB)),
            pltpu.SemaphoreType.DMA((N_DEV - 1, N_SUB)),
            pltpu.SemaphoreType.DMA((N_DEV - 1, N_SUB)),
            pltpu.SemaphoreType.DMA((N_DEV - 1, N_SUB)),
            pltpu.SemaphoreType.DMA,
            pltpu.SemaphoreType.DMA((2,)),
            pltpu.SemaphoreType.DMA((2,)),
        ],
        compiler_params=pltpu.CompilerParams(
            collective_id=0, vmem_limit_bytes=63 * 1024 * 1024
        ),
    )(x, w_mat, scale_x, scale_w)


# device time: 107825 ns/iter; 2.0298x vs baseline; 1.0259x over previous
import contextlib
import os

import jax
import jax.numpy as jnp
from jax import lax
from jax.experimental import pallas as pl
from jax.experimental.pallas import tpu as pltpu

N_DEV = 4
N_SUB = 2
_PROF = os.environ.get("KPROF") == "1"


def _scope(name):
    return jax.named_scope(name) if _PROF else contextlib.nullcontext()


def kernel(x, w_mat, scale_x, scale_w):
    m_per, k = x.shape
    n = w_mat.shape[1]
    n_per = n // N_DEV
    half = m_per // 2
    m_sub = half // N_SUB
    wq = 4
    nq = n_per // wq

    def body(x_hbm, w_hbm, sx_ref, sw_ref, out_hbm,
             xf_ref, x8_ref, wf_ref, w8_ref, comm_r, comm_l, ostage,
             send_r, recv_r, send_l, recv_l, xf_sem, wf_sem, os_sem):
        my_pos = lax.axis_index("i")
        left = (my_pos - 1) % N_DEV
        right = (my_pos + 1) % N_DEV

        xcp = [
            pltpu.make_async_copy(
                x_hbm.at[pl.ds(q * m_sub, m_sub)], xf_ref.at[q % 2],
                xf_sem.at[q % 2],
            )
            for q in range(4)
        ]
        xcp[0].start()

        barrier = pltpu.get_barrier_semaphore()
        for nbr in (left, right):
            pl.semaphore_signal(
                barrier, inc=1,
                device_id=(nbr,), device_id_type=pl.DeviceIdType.MESH,
            )
        pl.semaphore_wait(barrier, 2)

        def rdma(h, s, direction):
            rows = pl.ds(s * m_sub, m_sub)
            if direction == "r":
                src = (x8_ref.at[pl.ds(s * m_sub, m_sub)] if h == 0
                       else comm_r.at[h - 1, rows])
                return pltpu.make_async_remote_copy(
                    src_ref=src, dst_ref=comm_r.at[h, rows],
                    send_sem=send_r.at[h, s], recv_sem=recv_r.at[h, s],
                    device_id=(right,), device_id_type=pl.DeviceIdType.MESH,
                )
            src = (x8_ref.at[pl.ds(half + s * m_sub, m_sub)] if h == 0
                   else comm_l.at[h - 1, rows])
            return pltpu.make_async_remote_copy(
                src_ref=src, dst_ref=comm_l.at[h, rows],
                send_sem=send_l.at[h, s], recv_sem=recv_l.at[h, s],
                device_id=(left,), device_id_type=pl.DeviceIdType.MESH,
            )

        live = {}

        def start(h, s, d):
            r = rdma(h, s, d)
            r.start()
            live[(h, s, d)] = r

        with _scope("x_prep"):
            for q in range(4):
                if q < 3:
                    xcp[q + 1].start()
                xcp[q].wait()
                x8_ref[pl.ds(q * m_sub, m_sub), :] = (
                    xf_ref[q % 2].astype(jnp.float8_e4m3fn)
                )
                start(0, q % 2, "r" if q < 2 else "l")

        with _scope("w_prep"):
            wcp = [None, None]
            for q in range(wq):
                c = pltpu.make_async_copy(
                    w_hbm.at[:, pl.ds(my_pos * n_per + q * nq, nq)],
                    wf_ref.at[q % 2], wf_sem.at[q % 2],
                )
                c.start()
                wcp[q % 2] = c
                if q >= 1:
                    wcp[(q - 1) % 2].wait()
                    w8_ref[:, pl.ds((q - 1) * nq, nq)] = (
                        wf_ref[(q - 1) % 2].astype(jnp.float8_e5m2)
                    )
            wcp[(wq - 1) % 2].wait()
            w8_ref[:, pl.ds((wq - 1) * nq, nq)] = (
                wf_ref[(wq - 1) % 2].astype(jnp.float8_e5m2)
            )

        s_scale = sx_ref[0] * sw_ref[0]

        pending = [None, None]
        counter = [0]

        def block(a8, origin, off):
            acc = lax.dot_general(
                a8, w8_ref[...], (((1,), (0,)), ((), ())),
                preferred_element_type=jnp.float32,
            )
            y = acc * s_scale
            slot = counter[0] % 2
            if pending[slot] is not None:
                pending[slot].wait()
            ostage[slot, :, :] = y * jax.nn.sigmoid(y)
            c = pltpu.make_async_copy(
                ostage.at[slot],
                out_hbm.at[pl.ds(origin * m_per + off, m_sub)],
                os_sem.at[slot],
            )
            c.start()
            pending[slot] = c
            counter[0] += 1

        def sub_r(h, s):
            block(comm_r[h, pl.ds(s * m_sub, m_sub)],
                  (my_pos - h - 1) % N_DEV, s * m_sub)

        def sub_l(h, s):
            block(comm_l[h, pl.ds(s * m_sub, m_sub)],
                  (my_pos + h + 1) % N_DEV, half + s * m_sub)

        with _scope("gemm_own"):
            for r in range(4):
                block(x8_ref[pl.ds(r * m_sub, m_sub), :], my_pos, r * m_sub)

        for h in range(N_DEV - 2):
            for s in range(N_SUB):
                with _scope(f"wait_h{h}s{s}"):
                    live.pop((h, s, "r")).wait()
                    live.pop((h, s, "l")).wait()
                start(h + 1, s, "r")
                start(h + 1, s, "l")
                with _scope(f"gemm_h{h}s{s}"):
                    sub_r(h, s)
                    sub_l(h, s)
        h_last = N_DEV - 2
        for s in range(N_SUB):
            with _scope(f"wait_h{h_last}s{s}"):
                live.pop((h_last, s, "r")).wait()
                live.pop((h_last, s, "l")).wait()
            with _scope(f"gemm_h{h_last}s{s}"):
                sub_r(h_last, s)
                sub_l(h_last, s)
        with _scope("flush"):
            for p in pending:
                p.wait()

    return pl.pallas_call(
        body,
        out_shape=jax.ShapeDtypeStruct((N_DEV * m_per, n_per), jnp.float32),
        in_specs=[
            pl.BlockSpec(memory_space=pltpu.HBM),
            pl.BlockSpec(memory_space=pltpu.HBM),
            pl.BlockSpec(memory_space=pltpu.SMEM),
            pl.BlockSpec(memory_space=pltpu.SMEM),
        ],
        out_specs=pl.BlockSpec(memory_space=pltpu.HBM),
        scratch_shapes=[
            pltpu.VMEM((2, m_sub, k), jnp.float32),
            pltpu.VMEM((m_per, k), jnp.float8_e4m3fn),
            pltpu.VMEM((2, k, nq), jnp.float32),
            pltpu.VMEM((k, n_per), jnp.float8_e5m2),
            pltpu.VMEM((N_DEV - 1, half, k), jnp.float8_e4m3fn),
            pltpu.VMEM((N_DEV - 1, half, k), jnp.float8_e4m3fn),
            pltpu.VMEM((2, m_sub, n_per), jnp.float32),
            pltpu.SemaphoreType.DMA((N_DEV - 1, N_SUB)),
            pltpu.SemaphoreType.DMA((N_DEV - 1, N_SUB)),
            pltpu.SemaphoreType.DMA((N_DEV - 1, N_SUB)),
            pltpu.SemaphoreType.DMA((N_DEV - 1, N_SUB)),
            pltpu.SemaphoreType.DMA((2,)),
            pltpu.SemaphoreType.DMA((2,)),
            pltpu.SemaphoreType.DMA((2,)),
        ],
        compiler_params=pltpu.CompilerParams(
            collective_id=0, vmem_limit_bytes=63 * 1024 * 1024
        ),
    )(x, w_mat, scale_x, scale_w)


# device time: 106557 ns/iter; 2.0540x vs baseline; 1.0119x over previous
import contextlib
import os

import jax
import jax.numpy as jnp
from jax import lax
from jax.experimental import pallas as pl
from jax.experimental.pallas import tpu as pltpu

N_DEV = 4
N_SUB = 2
_PROF = os.environ.get("KPROF") == "1"


def _scope(name):
    return jax.named_scope(name) if _PROF else contextlib.nullcontext()


def kernel(x, w_mat, scale_x, scale_w):
    m_per, k = x.shape
    n = w_mat.shape[1]
    n_per = n // N_DEV
    half = m_per // 2
    m_sub = half // N_SUB
    wq = 4
    nq = n_per // wq

    def body(x_hbm, w_hbm, sx_ref, sw_ref, out_hbm,
             xf_ref, x8_ref, wf_ref, w8_ref, comm_r, comm_l, ostage,
             send_r, recv_r, send_l, recv_l, xf_sem, wf_sem, os_sem):
        my_pos = lax.axis_index("i")
        left = (my_pos - 1) % N_DEV
        right = (my_pos + 1) % N_DEV

        xcp = [
            pltpu.make_async_copy(
                x_hbm.at[pl.ds(q * m_sub, m_sub)], xf_ref.at[q % 2],
                xf_sem.at[q % 2],
            )
            for q in range(4)
        ]
        xcp[0].start()

        barrier = pltpu.get_barrier_semaphore()
        for nbr in (left, right):
            pl.semaphore_signal(
                barrier, inc=1,
                device_id=(nbr,), device_id_type=pl.DeviceIdType.MESH,
            )
        pl.semaphore_wait(barrier, 2)

        def rdma(h, s, direction):
            rows = pl.ds(s * m_sub, m_sub)
            if direction == "r":
                src = (x8_ref.at[pl.ds(s * m_sub, m_sub)] if h == 0
                       else comm_r.at[h - 1, rows])
                return pltpu.make_async_remote_copy(
                    src_ref=src, dst_ref=comm_r.at[h, rows],
                    send_sem=send_r.at[h, s], recv_sem=recv_r.at[h, s],
                    device_id=(right,), device_id_type=pl.DeviceIdType.MESH,
                )
            src = (x8_ref.at[pl.ds(half + s * m_sub, m_sub)] if h == 0
                   else comm_l.at[h - 1, rows])
            return pltpu.make_async_remote_copy(
                src_ref=src, dst_ref=comm_l.at[h, rows],
                send_sem=send_l.at[h, s], recv_sem=recv_l.at[h, s],
                device_id=(left,), device_id_type=pl.DeviceIdType.MESH,
            )

        live = {}

        def start(h, s, d):
            r = rdma(h, s, d)
            r.start()
            live[(h, s, d)] = r

        with _scope("x_prep"):
            for q in range(4):
                if q < 3:
                    xcp[q + 1].start()
                xcp[q].wait()
                x8_ref[pl.ds(q * m_sub, m_sub), :] = (
                    xf_ref[q % 2].astype(jnp.float8_e4m3fn)
                )
                start(0, q % 2, "r" if q < 2 else "l")

        with _scope("w_prep"):
            wcp = [None, None]
            for q in range(wq):
                c = pltpu.make_async_copy(
                    w_hbm.at[:, pl.ds(my_pos * n_per + q * nq, nq)],
                    wf_ref.at[q % 2], wf_sem.at[q % 2],
                )
                c.start()
                wcp[q % 2] = c
                if q >= 1:
                    wcp[(q - 1) % 2].wait()
                    w8_ref[:, pl.ds((q - 1) * nq, nq)] = (
                        wf_ref[(q - 1) % 2].astype(jnp.float8_e5m2)
                    )
            wcp[(wq - 1) % 2].wait()
            w8_ref[:, pl.ds((wq - 1) * nq, nq)] = (
                wf_ref[(wq - 1) % 2].astype(jnp.float8_e5m2)
            )

        s_scale = sx_ref[0] * sw_ref[0]

        pending = [None, None]
        counter = [0]

        def block(a8, origin, off):
            acc = lax.dot_general(
                a8, w8_ref[...], (((1,), (0,)), ((), ())),
                preferred_element_type=jnp.float32,
            )
            y = acc * s_scale
            slot = counter[0] % 2
            if pending[slot] is not None:
                pending[slot].wait()
            ostage[slot, :, :] = y * jax.nn.sigmoid(y)
            c = pltpu.make_async_copy(
                ostage.at[slot],
                out_hbm.at[pl.ds(origin * m_per + off, m_sub)],
                os_sem.at[slot],
            )
            c.start()
            pending[slot] = c
            counter[0] += 1

        def sub_r(h, s):
            block(comm_r[h, pl.ds(s * m_sub, m_sub)],
                  (my_pos - h - 1) % N_DEV, s * m_sub)

        def sub_l(h, s):
            block(comm_l[h, pl.ds(s * m_sub, m_sub)],
                  (my_pos + h + 1) % N_DEV, half + s * m_sub)

        with _scope("gemm_own"):
            for r in range(2):
                block(x8_ref[pl.ds(r * m_sub, m_sub), :], my_pos, r * m_sub)

        for h in range(N_DEV - 2):
            for s in range(N_SUB):
                with _scope(f"wait_h{h}s{s}"):
                    live.pop((h, s, "r")).wait()
                    live.pop((h, s, "l")).wait()
                start(h + 1, s, "r")
                start(h + 1, s, "l")
                if h == 0:
                    with _scope(f"gemm_own{2 + s}"):
                        block(x8_ref[pl.ds((2 + s) * m_sub, m_sub), :],
                              my_pos, (2 + s) * m_sub)
                with _scope(f"gemm_h{h}s{s}"):
                    sub_r(h, s)
                    sub_l(h, s)
        h_last = N_DEV - 2
        for s in range(N_SUB):
            with _scope(f"wait_h{h_last}s{s}"):
                live.pop((h_last, s, "r")).wait()
                live.pop((h_last, s, "l")).wait()
            with _scope(f"gemm_h{h_last}s{s}"):
                sub_r(h_last, s)
                sub_l(h_last, s)
        with _scope("flush"):
            for p in pending:
                p.wait()

    return pl.pallas_call(
        body,
        out_shape=jax.ShapeDtypeStruct((N_DEV * m_per, n_per), jnp.float32),
        in_specs=[
            pl.BlockSpec(memory_space=pltpu.HBM),
            pl.BlockSpec(memory_space=pltpu.HBM),
            pl.BlockSpec(memory_space=pltpu.SMEM),
            pl.BlockSpec(memory_space=pltpu.SMEM),
        ],
        out_specs=pl.BlockSpec(memory_space=pltpu.HBM),
        scratch_shapes=[
            pltpu.VMEM((2, m_sub, k), jnp.float32),
            pltpu.VMEM((m_per, k), jnp.float8_e4m3fn),
            pltpu.VMEM((2, k, nq), jnp.float32),
            pltpu.VMEM((k, n_per), jnp.float8_e5m2),
            pltpu.VMEM((N_DEV - 1, half, k), jnp.float8_e4m3fn),
            pltpu.VMEM((N_DEV - 1, half, k), jnp.float8_e4m3fn),
            pltpu.VMEM((2, m_sub, n_per), jnp.float32),
            pltpu.SemaphoreType.DMA((N_DEV - 1, N_SUB)),
            pltpu.SemaphoreType.DMA((N_DEV - 1, N_SUB)),
            pltpu.SemaphoreType.DMA((N_DEV - 1, N_SUB)),
            pltpu.SemaphoreType.DMA((N_DEV - 1, N_SUB)),
            pltpu.SemaphoreType.DMA((2,)),
            pltpu.SemaphoreType.DMA((2,)),
            pltpu.SemaphoreType.DMA((2,)),
        ],
        compiler_params=pltpu.CompilerParams(
            collective_id=0, vmem_limit_bytes=63 * 1024 * 1024
        ),
    )(x, w_mat, scale_x, scale_w)


# device time: 102005 ns/iter; 2.1456x vs baseline; 1.0446x over previous
import contextlib
import os

import jax
import jax.numpy as jnp
from jax import lax
from jax.experimental import pallas as pl
from jax.experimental.pallas import tpu as pltpu

N_DEV = 4
N_SUB = 2
_PROF = os.environ.get("KPROF") == "1"


def _scope(name):
    return jax.named_scope(name) if _PROF else contextlib.nullcontext()


def kernel(x, w_mat, scale_x, scale_w):
    m_per, k = x.shape
    n = w_mat.shape[1]
    n_per = n // N_DEV
    half = m_per // 2
    m_sub = half // N_SUB
    wq = 4
    nq = n_per // wq

    def body(x_hbm, w_hbm, sx_ref, sw_ref, out_hbm,
             xf_ref, x8_ref, wf_ref, w8_ref, comm_r, comm_l, ostage,
             send_r, recv_r, send_l, recv_l, xf_sem, wf_sem, os_sem):
        my_pos = lax.axis_index("i")
        left = (my_pos - 1) % N_DEV
        right = (my_pos + 1) % N_DEV

        xcp = [
            pltpu.make_async_copy(
                x_hbm.at[pl.ds(q * m_sub, m_sub)], xf_ref.at[q % 2],
                xf_sem.at[q % 2],
            )
            for q in range(4)
        ]
        xcp[0].start()

        barrier = pltpu.get_barrier_semaphore()
        for nbr in (left, right):
            pl.semaphore_signal(
                barrier, inc=1,
                device_id=(nbr,), device_id_type=pl.DeviceIdType.MESH,
            )
        pl.semaphore_wait(barrier, 2)

        def rdma(h, s, direction):
            rows = pl.ds(s * m_sub, m_sub)
            if direction == "r":
                src = (x8_ref.at[pl.ds(s * m_sub, m_sub)] if h == 0
                       else comm_r.at[h - 1, rows])
                return pltpu.make_async_remote_copy(
                    src_ref=src, dst_ref=comm_r.at[h, rows],
                    send_sem=send_r.at[h, s], recv_sem=recv_r.at[h, s],
                    device_id=(right,), device_id_type=pl.DeviceIdType.MESH,
                )
            src = (x8_ref.at[pl.ds(half + s * m_sub, m_sub)] if h == 0
                   else comm_l.at[h - 1, rows])
            return pltpu.make_async_remote_copy(
                src_ref=src, dst_ref=comm_l.at[h, rows],
                send_sem=send_l.at[h, s], recv_sem=recv_l.at[h, s],
                device_id=(left,), device_id_type=pl.DeviceIdType.MESH,
            )

        live = {}

        def start(h, s, d):
            r = rdma(h, s, d)
            r.start()
            live[(h, s, d)] = r

        with _scope("x_prep"):
            for q in range(4):
                if q < 3:
                    xcp[q + 1].start()
                xcp[q].wait()
                x8_ref[pl.ds(q * m_sub, m_sub), :] = (
                    xf_ref[q % 2].astype(jnp.float8_e4m3fn)
                )
                start(0, q % 2, "r" if q < 2 else "l")

        with _scope("w_prep"):
            wcp = [None, None]
            for q in range(wq):
                c = pltpu.make_async_copy(
                    w_hbm.at[:, pl.ds(my_pos * n_per + q * nq, nq)],
                    wf_ref.at[q % 2], wf_sem.at[q % 2],
                )
                c.start()
                wcp[q % 2] = c
                if q >= 1:
                    wcp[(q - 1) % 2].wait()
                    w8_ref[:, pl.ds((q - 1) * nq, nq)] = (
                        wf_ref[(q - 1) % 2].astype(jnp.float8_e5m2)
                    )
            wcp[(wq - 1) % 2].wait()
            w8_ref[:, pl.ds((wq - 1) * nq, nq)] = (
                wf_ref[(wq - 1) % 2].astype(jnp.float8_e5m2)
            )

        s_scale = sx_ref[0] * sw_ref[0]

        pending = [None, None]
        counter = [0]

        def block(a8, origin, off):
            acc = lax.dot_general(
                a8, w8_ref[...], (((1,), (0,)), ((), ())),
                preferred_element_type=jnp.float32,
            )
            y = acc * s_scale
            slot = counter[0] % 2
            if pending[slot] is not None:
                pending[slot].wait()
            ostage[slot, :, :] = (y * jax.nn.sigmoid(y)).astype(jnp.bfloat16)
            c = pltpu.make_async_copy(
                ostage.at[slot],
                out_hbm.at[pl.ds(origin * m_per + off, m_sub)],
                os_sem.at[slot],
            )
            c.start()
            pending[slot] = c
            counter[0] += 1

        def sub_r(h, s):
            block(comm_r[h, pl.ds(s * m_sub, m_sub)],
                  (my_pos - h - 1) % N_DEV, s * m_sub)

        def sub_l(h, s):
            block(comm_l[h, pl.ds(s * m_sub, m_sub)],
                  (my_pos + h + 1) % N_DEV, half + s * m_sub)

        with _scope("gemm_own"):
            for r in range(2):
                block(x8_ref[pl.ds(r * m_sub, m_sub), :], my_pos, r * m_sub)

        for h in range(N_DEV - 2):
            for s in range(N_SUB):
                with _scope(f"wait_h{h}s{s}"):
                    live.pop((h, s, "r")).wait()
                    live.pop((h, s, "l")).wait()
                start(h + 1, s, "r")
                start(h + 1, s, "l")
                if h == 0:
                    with _scope(f"gemm_own{2 + s}"):
                        block(x8_ref[pl.ds((2 + s) * m_sub, m_sub), :],
                              my_pos, (2 + s) * m_sub)
                with _scope(f"gemm_h{h}s{s}"):
                    sub_r(h, s)
                    sub_l(h, s)
        h_last = N_DEV - 2
        for s in range(N_SUB):
            with _scope(f"wait_h{h_last}s{s}"):
                live.pop((h_last, s, "r")).wait()
                live.pop((h_last, s, "l")).wait()
            with _scope(f"gemm_h{h_last}s{s}"):
                sub_r(h_last, s)
                sub_l(h_last, s)
        with _scope("flush"):
            for p in pending:
                p.wait()

    out16 = pl.pallas_call(
        body,
        out_shape=jax.ShapeDtypeStruct((N_DEV * m_per, n_per), jnp.bfloat16),
        in_specs=[
            pl.BlockSpec(memory_space=pltpu.HBM),
            pl.BlockSpec(memory_space=pltpu.HBM),
            pl.BlockSpec(memory_space=pltpu.SMEM),
            pl.BlockSpec(memory_space=pltpu.SMEM),
        ],
        out_specs=pl.BlockSpec(memory_space=pltpu.HBM),
        scratch_shapes=[
            pltpu.VMEM((2, m_sub, k), jnp.float32),
            pltpu.VMEM((m_per, k), jnp.float8_e4m3fn),
            pltpu.VMEM((2, k, nq), jnp.float32),
            pltpu.VMEM((k, n_per), jnp.float8_e5m2),
            pltpu.VMEM((N_DEV - 1, half, k), jnp.float8_e4m3fn),
            pltpu.VMEM((N_DEV - 1, half, k), jnp.float8_e4m3fn),
            pltpu.VMEM((2, m_sub, n_per), jnp.bfloat16),
            pltpu.SemaphoreType.DMA((N_DEV - 1, N_SUB)),
            pltpu.SemaphoreType.DMA((N_DEV - 1, N_SUB)),
            pltpu.SemaphoreType.DMA((N_DEV - 1, N_SUB)),
            pltpu.SemaphoreType.DMA((N_DEV - 1, N_SUB)),
            pltpu.SemaphoreType.DMA((2,)),
            pltpu.SemaphoreType.DMA((2,)),
            pltpu.SemaphoreType.DMA((2,)),
        ],
        compiler_params=pltpu.CompilerParams(
            collective_id=0, vmem_limit_bytes=63 * 1024 * 1024
        ),
    )(x, w_mat, scale_x, scale_w)
    return out16.astype(jnp.float32)


# device time: 100927 ns/iter; 2.1686x vs baseline; 1.0107x over previous
import contextlib
import os

import jax
import jax.numpy as jnp
from jax import lax
from jax.experimental import pallas as pl
from jax.experimental.pallas import tpu as pltpu

N_DEV = 4
N_SUB = 2
_PROF = os.environ.get("KPROF") == "1"


def _scope(name):
    return jax.named_scope(name) if _PROF else contextlib.nullcontext()


def kernel(x, w_mat, scale_x, scale_w):
    m_per, k = x.shape
    n = w_mat.shape[1]
    n_per = n // N_DEV
    half = m_per // 2
    m_sub = half // N_SUB
    m_q = m_sub // 2
    wq = 4
    nq = n_per // wq

    def body(x_hbm, w_hbm, sx_ref, sw_ref, out_hbm,
             xf_ref, x8_ref, wf_ref, w8_ref, comm_r, comm_l, ostage,
             send_r, recv_r, send_l, recv_l, xf_sem, wf_sem, os_sem):
        my_pos = lax.axis_index("i")
        left = (my_pos - 1) % N_DEV
        right = (my_pos + 1) % N_DEV

        xcp = [
            pltpu.make_async_copy(
                x_hbm.at[pl.ds(q * m_sub, m_sub)], xf_ref.at[q % 2],
                xf_sem.at[q % 2],
            )
            for q in range(4)
        ]
        xcp[0].start()

        barrier = pltpu.get_barrier_semaphore()
        for nbr in (left, right):
            pl.semaphore_signal(
                barrier, inc=1,
                device_id=(nbr,), device_id_type=pl.DeviceIdType.MESH,
            )
        pl.semaphore_wait(barrier, 2)

        def rdma(h, s, direction):
            sub = m_q if h == N_DEV - 2 else m_sub
            rows = pl.ds(s * sub, sub)
            if direction == "r":
                src = (x8_ref.at[pl.ds(s * m_sub, m_sub)] if h == 0
                       else comm_r.at[h - 1, rows])
                return pltpu.make_async_remote_copy(
                    src_ref=src, dst_ref=comm_r.at[h, rows],
                    send_sem=send_r.at[h, s], recv_sem=recv_r.at[h, s],
                    device_id=(right,), device_id_type=pl.DeviceIdType.MESH,
                )
            src = (x8_ref.at[pl.ds(half + s * m_sub, m_sub)] if h == 0
                   else comm_l.at[h - 1, rows])
            return pltpu.make_async_remote_copy(
                src_ref=src, dst_ref=comm_l.at[h, rows],
                send_sem=send_l.at[h, s], recv_sem=recv_l.at[h, s],
                device_id=(left,), device_id_type=pl.DeviceIdType.MESH,
            )

        live = {}

        def start(h, s, d):
            r = rdma(h, s, d)
            r.start()
            live[(h, s, d)] = r

        with _scope("x_prep"):
            for q in range(4):
                if q < 3:
                    xcp[q + 1].start()
                xcp[q].wait()
                x8_ref[pl.ds(q * m_sub, m_sub), :] = (
                    xf_ref[q % 2].astype(jnp.float8_e4m3fn)
                )
                start(0, q % 2, "r" if q < 2 else "l")

        with _scope("w_prep"):
            wcp = [None, None]
            for q in range(wq):
                c = pltpu.make_async_copy(
                    w_hbm.at[:, pl.ds(my_pos * n_per + q * nq, nq)],
                    wf_ref.at[q % 2], wf_sem.at[q % 2],
                )
                c.start()
                wcp[q % 2] = c
                if q >= 1:
                    wcp[(q - 1) % 2].wait()
                    w8_ref[:, pl.ds((q - 1) * nq, nq)] = (
                        wf_ref[(q - 1) % 2].astype(jnp.float8_e5m2)
                    )
            wcp[(wq - 1) % 2].wait()
            w8_ref[:, pl.ds((wq - 1) * nq, nq)] = (
                wf_ref[(wq - 1) % 2].astype(jnp.float8_e5m2)
            )

        s_scale = sx_ref[0] * sw_ref[0]

        pending = [None, None]
        counter = [0]

        def block(a8, origin, off, rows=None):
            rows = m_sub if rows is None else rows
            acc = lax.dot_general(
                a8, w8_ref[...], (((1,), (0,)), ((), ())),
                preferred_element_type=jnp.float32,
            )
            y = acc * s_scale
            slot = counter[0] % 2
            if pending[slot] is not None:
                pending[slot].wait()
            ostage[slot, pl.ds(0, rows), :] = (
                (y * jax.nn.sigmoid(y)).astype(jnp.bfloat16)
            )
            c = pltpu.make_async_copy(
                ostage.at[slot, pl.ds(0, rows)],
                out_hbm.at[pl.ds(origin * m_per + off, rows)],
                os_sem.at[slot],
            )
            c.start()
            pending[slot] = c
            counter[0] += 1

        def sub_r(h, s):
            block(comm_r[h, pl.ds(s * m_sub, m_sub)],
                  (my_pos - h - 1) % N_DEV, s * m_sub)

        def sub_l(h, s):
            block(comm_l[h, pl.ds(s * m_sub, m_sub)],
                  (my_pos + h + 1) % N_DEV, half + s * m_sub)

        with _scope("gemm_own"):
            for r in range(2):
                block(x8_ref[pl.ds(r * m_sub, m_sub), :], my_pos, r * m_sub)

        h_last = N_DEV - 2
        for h in range(N_DEV - 2):
            for s in range(N_SUB):
                with _scope(f"wait_h{h}s{s}"):
                    live.pop((h, s, "r")).wait()
                    live.pop((h, s, "l")).wait()
                if h + 1 == h_last:
                    for j in (2 * s, 2 * s + 1):
                        start(h_last, j, "r")
                        start(h_last, j, "l")
                else:
                    start(h + 1, s, "r")
                    start(h + 1, s, "l")
                if h == 0:
                    with _scope(f"gemm_own{2 + s}"):
                        block(x8_ref[pl.ds((2 + s) * m_sub, m_sub), :],
                              my_pos, (2 + s) * m_sub)
                with _scope(f"gemm_h{h}s{s}"):
                    sub_r(h, s)
                    sub_l(h, s)
        for j in range(2 * N_SUB):
            with _scope(f"wait_h{h_last}s{j}"):
                live.pop((h_last, j, "r")).wait()
                live.pop((h_last, j, "l")).wait()
            with _scope(f"gemm_h{h_last}s{j}"):
                block(comm_r[h_last, pl.ds(j * m_q, m_q)],
                      (my_pos - h_last - 1) % N_DEV, j * m_q, rows=m_q)
                block(comm_l[h_last, pl.ds(j * m_q, m_q)],
                      (my_pos + h_last + 1) % N_DEV, half + j * m_q, rows=m_q)
        with _scope("flush"):
            for p in pending:
                p.wait()

    out16 = pl.pallas_call(
        body,
        out_shape=jax.ShapeDtypeStruct((N_DEV * m_per, n_per), jnp.bfloat16),
        in_specs=[
            pl.BlockSpec(memory_space=pltpu.HBM),
            pl.BlockSpec(memory_space=pltpu.HBM),
            pl.BlockSpec(memory_space=pltpu.SMEM),
            pl.BlockSpec(memory_space=pltpu.SMEM),
        ],
        out_specs=pl.BlockSpec(memory_space=pltpu.HBM),
        scratch_shapes=[
            pltpu.VMEM((2, m_sub, k), jnp.float32),
            pltpu.VMEM((m_per, k), jnp.float8_e4m3fn),
            pltpu.VMEM((2, k, nq), jnp.float32),
            pltpu.VMEM((k, n_per), jnp.float8_e5m2),
            pltpu.VMEM((N_DEV - 1, half, k), jnp.float8_e4m3fn),
            pltpu.VMEM((N_DEV - 1, half, k), jnp.float8_e4m3fn),
            pltpu.VMEM((2, m_sub, n_per), jnp.bfloat16),
            pltpu.SemaphoreType.DMA((N_DEV - 1, 2 * N_SUB)),
            pltpu.SemaphoreType.DMA((N_DEV - 1, 2 * N_SUB)),
            pltpu.SemaphoreType.DMA((N_DEV - 1, 2 * N_SUB)),
            pltpu.SemaphoreType.DMA((N_DEV - 1, 2 * N_SUB)),
            pltpu.SemaphoreType.DMA((2,)),
            pltpu.SemaphoreType.DMA((2,)),
            pltpu.SemaphoreType.DMA((2,)),
        ],
        compiler_params=pltpu.CompilerParams(
            collective_id=0, vmem_limit_bytes=63 * 1024 * 1024
        ),
    )(x, w_mat, scale_x, scale_w)
    return out16.astype(jnp.float32)


# device time: 99129 ns/iter; 2.2079x vs baseline; 1.0181x over previous
import contextlib
import os

import jax
import jax.numpy as jnp
from jax import lax
from jax.experimental import pallas as pl
from jax.experimental.pallas import tpu as pltpu

N_DEV = 4
N_SUB = 2
_PROF = os.environ.get("KPROF") == "1"


def _scope(name):
    return jax.named_scope(name) if _PROF else contextlib.nullcontext()


def kernel(x, w_mat, scale_x, scale_w):
    m_per, k = x.shape
    n = w_mat.shape[1]
    n_per = n // N_DEV
    half = m_per // 2
    m_sub = half // N_SUB
    m_q = m_sub // 2
    wq = 4
    nq = n_per // wq

    def body(x_hbm, w_hbm, sx_ref, sw_ref, out_hbm,
             xf_ref, x8_ref, wf_ref, w8_ref, comm_r, comm_l, ostage,
             send_r, recv_r, send_l, recv_l, xf_sem, wf_sem, os_sem):
        my_pos = lax.axis_index("i")
        left = (my_pos - 1) % N_DEV
        right = (my_pos + 1) % N_DEV

        xcp = [
            pltpu.make_async_copy(
                x_hbm.at[pl.ds(q * m_sub, m_sub)], xf_ref.at[q % 2],
                xf_sem.at[q % 2],
            )
            for q in range(4)
        ]
        xcp[0].start()

        barrier = pltpu.get_barrier_semaphore()
        for nbr in (left, right):
            pl.semaphore_signal(
                barrier, inc=1,
                device_id=(nbr,), device_id_type=pl.DeviceIdType.MESH,
            )
        pl.semaphore_wait(barrier, 2)

        def rdma(h, s, direction):
            sub = m_q if h == N_DEV - 2 else m_sub
            rows = pl.ds(s * sub, sub)
            if direction == "r":
                src = (x8_ref.at[pl.ds(s * m_sub, m_sub)] if h == 0
                       else comm_r.at[h - 1, rows])
                return pltpu.make_async_remote_copy(
                    src_ref=src, dst_ref=comm_r.at[h, rows],
                    send_sem=send_r.at[h, s], recv_sem=recv_r.at[h, s],
                    device_id=(right,), device_id_type=pl.DeviceIdType.MESH,
                )
            src = (x8_ref.at[pl.ds(half + s * m_sub, m_sub)] if h == 0
                   else comm_l.at[h - 1, rows])
            return pltpu.make_async_remote_copy(
                src_ref=src, dst_ref=comm_l.at[h, rows],
                send_sem=send_l.at[h, s], recv_sem=recv_l.at[h, s],
                device_id=(left,), device_id_type=pl.DeviceIdType.MESH,
            )

        live = {}

        def start(h, s, d):
            r = rdma(h, s, d)
            r.start()
            live[(h, s, d)] = r

        with _scope("x_prep"):
            for q in range(4):
                if q < 3:
                    xcp[q + 1].start()
                xcp[q].wait()
                x8_ref[pl.ds(q * m_sub, m_sub), :] = (
                    xf_ref[q % 2].astype(jnp.float8_e4m3fn)
                )
                start(0, q % 2, "r" if q < 2 else "l")

        with _scope("w_prep"):
            wcp = [None, None]
            for q in range(wq):
                c = pltpu.make_async_copy(
                    w_hbm.at[:, pl.ds(my_pos * n_per + q * nq, nq)],
                    wf_ref.at[q % 2], wf_sem.at[q % 2],
                )
                c.start()
                wcp[q % 2] = c
                if q >= 1:
                    wcp[(q - 1) % 2].wait()
                    w8_ref[:, pl.ds((q - 1) * nq, nq)] = (
                        wf_ref[(q - 1) % 2].astype(jnp.float8_e5m2)
                    )
            wcp[(wq - 1) % 2].wait()
            w8_ref[:, pl.ds((wq - 1) * nq, nq)] = (
                wf_ref[(wq - 1) % 2].astype(jnp.float8_e5m2)
            )

        s_scale = sx_ref[0] * sw_ref[0]

        pending = [None, None]
        counter = [0]

        def block(a8, origin, off, rows=None):
            rows = m_sub if rows is None else rows
            acc = lax.dot_general(
                a8, w8_ref[...], (((1,), (0,)), ((), ())),
                preferred_element_type=jnp.float32,
            )
            y = acc * s_scale
            slot = counter[0] % 2
            if pending[slot] is not None:
                pending[slot].wait()
            ostage[slot, pl.ds(0, rows), :] = (
                (y * jax.nn.sigmoid(y)).astype(jnp.bfloat16)
            )
            c = pltpu.make_async_copy(
                ostage.at[slot, pl.ds(0, rows)],
                out_hbm.at[pl.ds(origin * m_per + off, rows)],
                os_sem.at[slot],
            )
            c.start()
            pending[slot] = c
            counter[0] += 1

        def sub_r(h, s):
            block(comm_r[h, pl.ds(s * m_sub, m_sub)],
                  (my_pos - h - 1) % N_DEV, s * m_sub)

        def sub_l(h, s):
            block(comm_l[h, pl.ds(s * m_sub, m_sub)],
                  (my_pos + h + 1) % N_DEV, half + s * m_sub)

        with _scope("gemm_own"):
            for r in range(2):
                block(x8_ref[pl.ds(r * m_sub, m_sub), :], my_pos, r * m_sub)

        h_last = N_DEV - 2
        for h in range(N_DEV - 2):
            for s in range(N_SUB):
                with _scope(f"wait_h{h}s{s}"):
                    live.pop((h, s, "r")).wait()
                    live.pop((h, s, "l")).wait()
                if h + 1 == h_last:
                    for j in (2 * s, 2 * s + 1):
                        start(h_last, j, "r")
                        start(h_last, j, "l")
                else:
                    start(h + 1, s, "r")
                    start(h + 1, s, "l")
                if h == 0:
                    with _scope(f"gemm_own{2 + s}"):
                        block(x8_ref[pl.ds((2 + s) * m_sub, m_sub), :],
                              my_pos, (2 + s) * m_sub)
                with _scope(f"gemm_h{h}s{s}"):
                    sub_r(h, s)
                    sub_l(h, s)
        for j in range(2 * N_SUB):
            with _scope(f"wait_h{h_last}s{j}"):
                live.pop((h_last, j, "r")).wait()
                live.pop((h_last, j, "l")).wait()
            with _scope(f"gemm_h{h_last}s{j}"):
                block(comm_r[h_last, pl.ds(j * m_q, m_q)],
                      (my_pos - h_last - 1) % N_DEV, j * m_q, rows=m_q)
                block(comm_l[h_last, pl.ds(j * m_q, m_q)],
                      (my_pos + h_last + 1) % N_DEV, half + j * m_q, rows=m_q)
        with _scope("flush"):
            for p in pending:
                p.wait()

    out16 = pl.pallas_call(
        body,
        out_shape=jax.ShapeDtypeStruct((N_DEV * m_per, n_per), jnp.bfloat16),
        in_specs=[
            pl.BlockSpec(memory_space=pltpu.HBM),
            pl.BlockSpec(memory_space=pltpu.HBM),
            pl.BlockSpec(memory_space=pltpu.SMEM),
            pl.BlockSpec(memory_space=pltpu.SMEM),
        ],
        out_specs=pl.BlockSpec(memory_space=pltpu.HBM),
        scratch_shapes=[
            pltpu.VMEM((2, m_sub, k), jnp.float32),
            pltpu.VMEM((m_per, k), jnp.float8_e4m3fn),
            pltpu.VMEM((2, k, nq), jnp.float32),
            pltpu.VMEM((k, n_per), jnp.float8_e5m2),
            pltpu.VMEM((N_DEV - 1, half, k), jnp.float8_e4m3fn),
            pltpu.VMEM((N_DEV - 1, half, k), jnp.float8_e4m3fn),
            pltpu.VMEM((2, m_sub, n_per), jnp.bfloat16),
            pltpu.SemaphoreType.DMA((N_DEV - 1, 2 * N_SUB)),
            pltpu.SemaphoreType.DMA((N_DEV - 1, 2 * N_SUB)),
            pltpu.SemaphoreType.DMA((N_DEV - 1, 2 * N_SUB)),
            pltpu.SemaphoreType.DMA((N_DEV - 1, 2 * N_SUB)),
            pltpu.SemaphoreType.DMA((2,)),
            pltpu.SemaphoreType.DMA((2,)),
            pltpu.SemaphoreType.DMA((2,)),
        ],
        compiler_params=pltpu.CompilerParams(
            collective_id=0, vmem_limit_bytes=63 * 1024 * 1024
        ),
    )(x, w_mat, scale_x, scale_w)
    def conv_body(i_ref, o_ref):
        o_ref[...] = i_ref[...].astype(jnp.float32)

    blk = 512
    return pl.pallas_call(
        conv_body,
        grid=(N_DEV * m_per // blk,),
        in_specs=[pl.BlockSpec((blk, n_per), lambda i: (i, 0))],
        out_specs=pl.BlockSpec((blk, n_per), lambda i: (i, 0)),
        out_shape=jax.ShapeDtypeStruct((N_DEV * m_per, n_per), jnp.float32),
    )(out16)
